# baseline (speedup 1.0000x reference)
import functools

import jax
import jax.numpy as jnp
from jax import lax
from jax.experimental import pallas as pl
from jax.experimental.pallas import tpu as pltpu
from jax.experimental.pallas import tpu_sc as plsc

_NC = 2
_NS = 16
_L = 16
_NW = _NC * _NS

_F = 16


def _sc_mesh():
    return plsc.VectorSubcoreMesh(
        core_axis_name="c", subcore_axis_name="s",
        num_cores=_NC, num_subcores=_NS)


def _rpt(n):
    return -(-(n + 1) // (8 * _NS)) * 8


def _pad_edges(src, dst, n):
    e = src.shape[0]
    rows = -(-e // 128)
    rps = -(-(-(-rows // _NW)) // 8) * 8
    tot = rps * _NW * 128
    srcp = jnp.concatenate([src, jnp.zeros((tot - e,), jnp.int32)])
    dstp = jnp.concatenate([dst, jnp.full((tot - e,), n, jnp.int32)])
    return srcp.reshape(-1, 128), dstp.reshape(-1, 128), rps


def _chunks(rpt):
    out = []
    off = 0
    while off < rpt:
        sz = min(128, rpt - off)
        out.append((off, sz))
        off += sz
    return out


def _seg128(vals, src2, dst2, rps):
    n = vals.shape[0]
    rpt = _rpt(n)
    npad = rpt * _NS

    @functools.partial(
        pl.kernel,
        out_type=jax.ShapeDtypeStruct((_NC, npad, 128), jnp.float32),
        mesh=_sc_mesh(),
        scratch_types=[
            pltpu.VMEM((rps // 2, 128), jnp.int32),
            pltpu.VMEM((rps // 2, 128), jnp.int32),
            pltpu.VMEM((128, 128), jnp.float32),
            pltpu.VMEM((128, 128), jnp.float32),
            pltpu.VMEM_SHARED((npad, 128), jnp.float32),
            pltpu.SemaphoreType.DMA,
        ],
    )
    def run(vals_hbm, src_hbm, dst_hbm, out_hbm, idxs, idxd,
            b0, b1, acc, gsem):
        cid = lax.axis_index("c")
        sid = lax.axis_index("s")
        wid = cid * _NS + sid
        bufs = (b0, b1)
        nbuf = len(bufs)

        @pl.loop(0, 128)
        def _zero(i):
            for j in range(8):
                b0[i, pl.ds(j * _L, _L)] = jnp.zeros((_L,), jnp.float32)

        for off, sz in _chunks(rpt):
            pltpu.sync_copy(b0.at[pl.ds(0, sz)],
                            acc.at[pl.ds(sid * rpt + off, sz)])
        plsc.subcore_barrier()

        hp2 = rps // 2
        for ph in range(2):
            pltpu.sync_copy(src_hbm.at[pl.ds(wid * rps + ph * hp2, hp2)], idxs)
            pltpu.sync_copy(dst_hbm.at[pl.ds(wid * rps + ph * hp2, hp2)], idxd)

            @pl.loop(0, hp2 // nbuf)
            def _edges(k):
                j = k * nbuf
                gh = [pltpu.async_copy(vals_hbm.at[idxs.at[j + b]],
                                       bufs[b], gsem)
                      for b in range(nbuf)]
                for b in range(nbuf):
                    gh[b].wait()
                    pltpu.sync_copy(bufs[b], acc.at[idxd.at[j + b]], add=True)

        plsc.subcore_barrier()
        for off, sz in _chunks(rpt):
            pltpu.sync_copy(acc.at[pl.ds(sid * rpt + off, sz)],
                            b0.at[pl.ds(0, sz)])
            pltpu.sync_copy(b0.at[pl.ds(0, sz)],
                            out_hbm.at[cid, pl.ds(sid * rpt + off, sz)])

    return run(vals, src2, dst2)


def _segsc(vals, src2, dst2, rps):
    n = vals.shape[0]
    rpt = _rpt(n)
    npad = rpt * _NS

    @functools.partial(
        pl.kernel,
        out_type=jax.ShapeDtypeStruct((_NC * npad,), jnp.float32),
        mesh=_sc_mesh(),
        scratch_types=[
            pltpu.VMEM((rps * 128,), jnp.int32),
            pltpu.VMEM((rps, 128), jnp.int32),
            pltpu.VMEM((1024,), jnp.float32),
            pltpu.VMEM((1024,), jnp.float32),
            pltpu.VMEM((128,), jnp.float32),
            pltpu.VMEM_SHARED((npad,), jnp.float32),
            pltpu.SemaphoreType.DMA,
        ],
    )
    def run(vals_hbm, srcf_hbm, dst_hbm, out_hbm, idxs, idxd,
            b0, b1, vc, acc, gsem):
        cid = lax.axis_index("c")
        sid = lax.axis_index("s")
        wid = cid * _NS + sid
        bufs = (b0, b1)
        nbuf = len(bufs)

        for j in range(8):
            vc[pl.ds(j * _L, _L)] = jnp.zeros((_L,), jnp.float32)
        for off, sz in _chunks(rpt):
            pltpu.sync_copy(vc.at[pl.ds(0, sz)],
                            acc.at[pl.ds(sid * rpt + off, sz)])
        plsc.subcore_barrier()

        pltpu.sync_copy(srcf_hbm.at[pl.ds(wid * rps * 128, rps * 128)], idxs)
        pltpu.sync_copy(dst_hbm.at[pl.ds(wid * rps, rps)], idxd)
        ng = rps * 128 // 1024

        @pl.loop(0, ng // nbuf)
        def _edges(k):
            g = k * nbuf
            gh = [pltpu.async_copy(
                      vals_hbm.at[idxs.at[pl.ds((g + b) * 1024, 1024)]],
                      bufs[b], gsem)
                  for b in range(nbuf)]
            for b in range(nbuf):
                gh[b].wait()
                for i in range(8):
                    pltpu.sync_copy(
                        bufs[b].at[pl.ds(i * 128, 128)],
                        acc.at[idxd.at[(g + b) * 8 + i]], add=True)

        plsc.subcore_barrier()
        for off, sz in _chunks(rpt):
            pltpu.sync_copy(acc.at[pl.ds(sid * rpt + off, sz)],
                            vc.at[pl.ds(0, sz)])
            pltpu.sync_copy(
                vc.at[pl.ds(0, sz)],
                out_hbm.at[pl.ds(cid * npad + sid * rpt + off, sz)])

    return run(vals, src2.reshape(-1), dst2).reshape(_NC, npad)


def _seghist(dst2, rps, n):
    rpt = _rpt(n)
    npad = rpt * _NS

    @functools.partial(
        pl.kernel,
        out_type=jax.ShapeDtypeStruct((_NC * npad,), jnp.float32),
        mesh=_sc_mesh(),
        scratch_types=[
            pltpu.VMEM((rps, 128), jnp.int32),
            pltpu.VMEM((128,), jnp.float32),
            pltpu.VMEM_SHARED((npad,), jnp.float32),
        ],
    )
    def run(dst_hbm, out_hbm, idxd, vc, acc):
        cid = lax.axis_index("c")
        sid = lax.axis_index("s")
        wid = cid * _NS + sid

        for j in range(8):
            vc[pl.ds(j * _L, _L)] = jnp.zeros((_L,), jnp.float32)
        for off, sz in _chunks(rpt):
            pltpu.sync_copy(vc.at[pl.ds(0, sz)],
                            acc.at[pl.ds(sid * rpt + off, sz)])
        plsc.subcore_barrier()

        for j in range(8):
            vc[pl.ds(j * _L, _L)] = jnp.ones((_L,), jnp.float32)
        pltpu.sync_copy(dst_hbm.at[pl.ds(wid * rps, rps)], idxd)

        @pl.loop(0, rps)
        def _edges(j):
            pltpu.sync_copy(vc, acc.at[idxd.at[j]], add=True)

        plsc.subcore_barrier()
        for j in range(8):
            vc[pl.ds(j * _L, _L)] = jnp.zeros((_L,), jnp.float32)
        for off, sz in _chunks(rpt):
            pltpu.sync_copy(acc.at[pl.ds(sid * rpt + off, sz)],
                            vc.at[pl.ds(0, sz)])
            pltpu.sync_copy(
                vc.at[pl.ds(0, sz)],
                out_hbm.at[pl.ds(cid * npad + sid * rpt + off, sz)])

    return run(dst2).reshape(_NC, npad)


def _place(v16):
    p = (lax.broadcasted_iota(jnp.int32, (_F, 128), 0)
         == lax.broadcasted_iota(jnp.int32, (_F, 128), 1)).astype(jnp.float32)
    return jnp.dot(v16, p, preferred_element_type=jnp.float32)


def _tc_a(x, w1, degp):
    n, d = x.shape
    f = w1.shape[1]
    rb = 1000
    g = n // rb

    def body(x_ref, w_ref, dp_ref, h_ref, hp_ref, dinv_ref):
        deg = dp_ref[0] + dp_ref[1] + 1.0
        dinv = lax.rsqrt(deg)
        h = jnp.dot(x_ref[...], w_ref[...], preferred_element_type=jnp.float32)
        h_ref[...] = h
        hp_ref[...] = _place(h * dinv)
        dinv_ref[...] = dinv

    return pl.pallas_call(
        body,
        grid=(g,),
        in_specs=[
            pl.BlockSpec((rb, d), lambda i: (i, 0)),
            pl.BlockSpec((d, f), lambda i: (0, 0)),
            pl.BlockSpec((2, rb, 1), lambda i: (0, i, 0)),
        ],
        out_specs=[
            pl.BlockSpec((rb, f), lambda i: (i, 0)),
            pl.BlockSpec((rb, 128), lambda i: (i, 0)),
            pl.BlockSpec((rb, 1), lambda i: (i, 0)),
        ],
        out_shape=[
            jax.ShapeDtypeStruct((n, f), jnp.float32),
            jax.ShapeDtypeStruct((n, 128), jnp.float32),
            jax.ShapeDtypeStruct((n, 1), jnp.float32),
        ],
    )(x, w1, degp)


def _tc_c(s1p, dinv1, h, b1r, wnr, wrr):
    n, f = h.shape
    rb = 1000
    g = n // rb

    def body(s_ref, dinv_ref, h_ref, b1_ref, wn_ref, wr_ref,
             h1_ref, u_ref, v_ref):
        dinv = dinv_ref[...]
        agg = dinv * (s_ref[0] + s_ref[1]) + dinv * dinv * h_ref[...] + b1_ref[...]
        h1 = jnp.maximum(agg, 0.0)
        h1_ref[...] = h1
        u_ref[...] = jnp.sum(h1 * wn_ref[...], axis=1, keepdims=True)
        v_ref[...] = jnp.sum(h1 * wr_ref[...], axis=1, keepdims=True)

    return pl.pallas_call(
        body,
        grid=(g,),
        in_specs=[
            pl.BlockSpec((2, rb, f), lambda i: (0, i, 0)),
            pl.BlockSpec((rb, 1), lambda i: (i, 0)),
            pl.BlockSpec((rb, f), lambda i: (i, 0)),
            pl.BlockSpec((1, f), lambda i: (0, 0)),
            pl.BlockSpec((1, f), lambda i: (0, 0)),
            pl.BlockSpec((1, f), lambda i: (0, 0)),
        ],
        out_specs=[
            pl.BlockSpec((rb, f), lambda i: (i, 0)),
            pl.BlockSpec((rb, 1), lambda i: (i, 0)),
            pl.BlockSpec((rb, 1), lambda i: (i, 0)),
        ],
        out_shape=[
            jax.ShapeDtypeStruct((n, f), jnp.float32),
            jax.ShapeDtypeStruct((n, 1), jnp.float32),
            jax.ShapeDtypeStruct((n, 1), jnp.float32),
        ],
    )(s1p, dinv1, h, b1r, wnr, wrr)


def _tc_d(sac, sar, vcol, vrow, bcol, brow, h1, bsr, nb):
    n, f = h1.shape
    rb = 400
    g = n // rb
    cj = 1000
    ncj = n // cj

    def body(sac_ref, sar_ref, vc_ref, vr_ref, bc_ref, br_ref, h1_ref,
             bs_ref, y_ref, kf_ref, kv_ref):
        i = pl.program_id(0)
        bsv = bs_ref[...]
        score_i = sac_ref[0] + sac_ref[1] + vc_ref[...] + bsv
        srow = sar_ref[0] + sar_ref[1] + vr_ref[...] + bsv
        brow_v = br_ref[...]
        bcol_v = bc_ref[...]

        iota_g = lax.broadcasted_iota(jnp.int32, (nb, 1), 0)
        counts = jnp.sum((brow_v == iota_g).astype(jnp.float32),
                         axis=1, keepdims=True)
        kv = jnp.ceil(jnp.float32(0.8) * counts)
        kv_ref[...] = kv

        iidx = i * rb + lax.broadcasted_iota(jnp.int32, (rb, 1), 0)
        rank = jnp.zeros((rb, 1), jnp.float32)
        for c in range(ncj):
            sj = srow[:, c * cj:(c + 1) * cj]
            bj = brow_v[:, c * cj:(c + 1) * cj]
            jidx = c * cj + lax.broadcasted_iota(jnp.int32, (1, cj), 1)
            hit = ((bj == bcol_v)
                   & ((sj > score_i)
                      | ((sj == score_i) & (jidx < iidx))))
            rank = rank + jnp.sum(hit.astype(jnp.float32),
                                  axis=1, keepdims=True)

        ohr = (bcol_v == lax.broadcasted_iota(jnp.int32, (1, nb), 1)
               ).astype(jnp.float32)
        kb = jnp.dot(ohr, kv, preferred_element_type=jnp.float32)
        kf = (rank < kb).astype(jnp.float32)
        kf_ref[...] = kf
        y_ref[...] = kf * h1_ref[...] * jnp.tanh(score_i)

    return pl.pallas_call(
        body,
        grid=(g,),
        in_specs=[
            pl.BlockSpec((2, rb, 1), lambda i: (0, i, 0)),
            pl.BlockSpec((2, 1, n), lambda i: (0, 0, 0)),
            pl.BlockSpec((rb, 1), lambda i: (i, 0)),
            pl.BlockSpec((1, n), lambda i: (0, 0)),
            pl.BlockSpec((rb, 1), lambda i: (i, 0)),
            pl.BlockSpec((1, n), lambda i: (0, 0)),
            pl.BlockSpec((rb, f), lambda i: (i, 0)),
            pl.BlockSpec((1, 1), lambda i: (0, 0)),
        ],
        out_specs=[
            pl.BlockSpec((rb, f), lambda i: (i, 0)),
            pl.BlockSpec((rb, 1), lambda i: (i, 0)),
            pl.BlockSpec((nb, 1), lambda i: (0, 0)),
        ],
        out_shape=[
            jax.ShapeDtypeStruct((n, f), jnp.float32),
            jax.ShapeDtypeStruct((n, 1), jnp.float32),
            jax.ShapeDtypeStruct((nb, 1), jnp.float32),
        ],
    )(sac, sar, vcol, vrow, bcol, brow, h1, bsr)


def _tc_e(d2p, y):
    n, f = y.shape
    rb = 1000
    g = n // rb

    def body(dp_ref, y_ref, dinv_ref, yp_ref):
        deg = dp_ref[0] + dp_ref[1] + 1.0
        dinv = lax.rsqrt(deg)
        dinv_ref[...] = dinv
        yp_ref[...] = _place(y_ref[...] * dinv)

    return pl.pallas_call(
        body,
        grid=(g,),
        in_specs=[
            pl.BlockSpec((2, rb, 1), lambda i: (0, i, 0)),
            pl.BlockSpec((rb, f), lambda i: (i, 0)),
        ],
        out_specs=[
            pl.BlockSpec((rb, 1), lambda i: (i, 0)),
            pl.BlockSpec((rb, 128), lambda i: (i, 0)),
        ],
        out_shape=[
            jax.ShapeDtypeStruct((n, 1), jnp.float32),
            jax.ShapeDtypeStruct((n, 128), jnp.float32),
        ],
    )(d2p, y)


def _tc_f(s2p, dinv2, y, keptf, w2, b2r, brow, kv, nb):
    n, f = y.shape
    hid = w2.shape[1]
    rb = 1000
    g = n // rb

    def body(s2_ref, dinv_ref, y_ref, kf_ref, w2_ref, b2_ref, br_ref,
             kv_ref, out_ref):
        i = pl.program_id(0)
        dinv = dinv_ref[...]
        agg16 = dinv * (s2_ref[0] + s2_ref[1]) + dinv * dinv * y_ref[...]
        pre = jnp.dot(agg16, w2_ref[...],
                      preferred_element_type=jnp.float32) + b2_ref[...]
        h2 = kf_ref[...] * jnp.maximum(pre, 0.0)
        oh = (br_ref[...] == lax.broadcasted_iota(jnp.int32, (1, nb), 1)
              ).astype(jnp.float32)
        contrib = lax.dot_general(
            oh, h2, (((0,), (0,)), ((), ())),
            preferred_element_type=jnp.float32)

        @pl.when(i == 0)
        def _():
            out_ref[...] = jnp.zeros_like(out_ref)

        out_ref[...] += contrib

        @pl.when(i == g - 1)
        def _():
            out_ref[...] = out_ref[...] / jnp.maximum(kv_ref[...], 1.0)

    return pl.pallas_call(
        body,
        grid=(g,),
        in_specs=[
            pl.BlockSpec((2, rb, f), lambda i: (0, i, 0)),
            pl.BlockSpec((rb, 1), lambda i: (i, 0)),
            pl.BlockSpec((rb, f), lambda i: (i, 0)),
            pl.BlockSpec((rb, 1), lambda i: (i, 0)),
            pl.BlockSpec((f, hid), lambda i: (0, 0)),
            pl.BlockSpec((1, hid), lambda i: (0, 0)),
            pl.BlockSpec((rb, 1), lambda i: (i, 0)),
            pl.BlockSpec((nb, 1), lambda i: (0, 0)),
        ],
        out_specs=pl.BlockSpec((nb, hid), lambda i: (0, 0)),
        out_shape=jax.ShapeDtypeStruct((nb, hid), jnp.float32),
    )(s2p, dinv2, y, keptf, w2, b2r, brow, kv)


def kernel(x, edge_index, batch, W1, b1, Wr, Wn, bs, W2, b2):
    n = x.shape[0]
    nb = 64
    f = W1.shape[1]
    hid = W2.shape[1]
    src2, dst2, rps = _pad_edges(edge_index[0], edge_index[1], n)

    deg1p = _seghist(dst2, rps, n)[:, :n]
    h, hp128, dinv1 = _tc_a(x, W1, deg1p.reshape(2, n, 1))

    s1p = _seg128(hp128, src2, dst2, rps)[:, :n, :f]
    h1, u, v = _tc_c(s1p, dinv1, h, b1.reshape(1, f),
                     Wn.reshape(1, f), Wr.reshape(1, f))

    sap = _segsc(u.reshape(n), src2, dst2, rps)[:, :n]
    y, keptf, kv = _tc_d(sap.reshape(2, n, 1), sap.reshape(2, 1, n),
                         v, v.reshape(1, n),
                         batch.reshape(n, 1), batch.reshape(1, n),
                         h1, bs.reshape(1, 1), nb)

    d2p = _segsc(keptf.reshape(n), src2, dst2, rps)[:, :n]
    dinv2, yp128 = _tc_e(d2p.reshape(2, n, 1), y)

    s2p = _seg128(yp128, src2, dst2, rps)[:, :n, :f]
    return _tc_f(s2p, dinv2, y, keptf, W2, b2.reshape(1, hid),
                 batch.reshape(n, 1), kv, nb)


# speedup vs baseline: 1.0215x; 1.0215x over previous
import functools

import jax
import jax.numpy as jnp
from jax import lax
from jax.experimental import pallas as pl
from jax.experimental.pallas import tpu as pltpu
from jax.experimental.pallas import tpu_sc as plsc

_NC = 2
_NS = 16
_L = 16
_NW = _NC * _NS

_F = 16


def _sc_mesh():
    return plsc.VectorSubcoreMesh(
        core_axis_name="c", subcore_axis_name="s",
        num_cores=_NC, num_subcores=_NS)


def _rpt(n):
    return -(-(n + 1) // (8 * _NS)) * 8


def _pad_edges(src, dst, n):
    e = src.shape[0]
    rows = -(-e // 128)
    rps = -(-(-(-rows // _NW)) // 8) * 8
    tot = rps * _NW * 128
    srcp = jnp.concatenate([src, jnp.zeros((tot - e,), jnp.int32)])
    dstp = jnp.concatenate([dst, jnp.full((tot - e,), n, jnp.int32)])
    return srcp.reshape(-1, 128), dstp.reshape(-1, 128), rps


def _chunks(rpt):
    out = []
    off = 0
    while off < rpt:
        sz = min(128, rpt - off)
        out.append((off, sz))
        off += sz
    return out


def _seg128(vals, src2, dst2, rps):
    n = vals.shape[0]
    rpt = _rpt(n)
    npad = rpt * _NS

    @functools.partial(
        pl.kernel,
        out_type=jax.ShapeDtypeStruct((_NC, npad, 128), jnp.float32),
        mesh=_sc_mesh(),
        scratch_types=[
            pltpu.VMEM((rps // 2, 128), jnp.int32),
            pltpu.VMEM((rps // 2, 128), jnp.int32),
            pltpu.VMEM((128, 128), jnp.float32),
            pltpu.VMEM((128, 128), jnp.float32),
            pltpu.VMEM_SHARED((npad, 128), jnp.float32),
            pltpu.SemaphoreType.DMA,
        ],
    )
    def run(vals_hbm, src_hbm, dst_hbm, out_hbm, idxs, idxd,
            b0, b1, acc, gsem):
        cid = lax.axis_index("c")
        sid = lax.axis_index("s")
        wid = cid * _NS + sid
        bufs = (b0, b1)
        nbuf = len(bufs)

        @pl.loop(0, 128)
        def _zero(i):
            for j in range(8):
                b0[i, pl.ds(j * _L, _L)] = jnp.zeros((_L,), jnp.float32)

        for off, sz in _chunks(rpt):
            pltpu.sync_copy(b0.at[pl.ds(0, sz)],
                            acc.at[pl.ds(sid * rpt + off, sz)])
        plsc.subcore_barrier()

        hp2 = rps // 2
        for ph in range(2):
            pltpu.sync_copy(src_hbm.at[pl.ds(wid * rps + ph * hp2, hp2)], idxs)
            pltpu.sync_copy(dst_hbm.at[pl.ds(wid * rps + ph * hp2, hp2)], idxd)

            @pl.loop(0, hp2 // nbuf)
            def _edges(k):
                j = k * nbuf
                gh = [pltpu.async_copy(vals_hbm.at[idxs.at[j + b]],
                                       bufs[b], gsem)
                      for b in range(nbuf)]
                for b in range(nbuf):
                    gh[b].wait()
                    pltpu.sync_copy(bufs[b], acc.at[idxd.at[j + b]], add=True)

        plsc.subcore_barrier()
        for off, sz in _chunks(rpt):
            pltpu.sync_copy(acc.at[pl.ds(sid * rpt + off, sz)],
                            b0.at[pl.ds(0, sz)])
            pltpu.sync_copy(b0.at[pl.ds(0, sz)],
                            out_hbm.at[cid, pl.ds(sid * rpt + off, sz)])

    return run(vals, src2, dst2)


def _segsc(vals, src2, dst2, rps):
    n = vals.shape[0]
    rpt = _rpt(n)
    npad = rpt * _NS

    @functools.partial(
        pl.kernel,
        out_type=jax.ShapeDtypeStruct((_NC * npad,), jnp.float32),
        mesh=_sc_mesh(),
        scratch_types=[
            pltpu.VMEM((rps // 2, 128), jnp.int32),
            pltpu.VMEM((rps // 2, 128), jnp.int32),
            pltpu.VMEM((128,), jnp.float32),
            pltpu.VMEM((128,), jnp.float32),
            pltpu.VMEM_SHARED((npad,), jnp.float32),
            pltpu.SemaphoreType.DMA,
        ],
    )
    def run(vals_hbm, src_hbm, dst_hbm, out_hbm, idxs, idxd,
            b0, b1, acc, gsem):
        cid = lax.axis_index("c")
        sid = lax.axis_index("s")
        wid = cid * _NS + sid
        bufs = (b0, b1)
        nbuf = len(bufs)

        for j in range(8):
            b0[pl.ds(j * _L, _L)] = jnp.zeros((_L,), jnp.float32)
        for off, sz in _chunks(rpt):
            pltpu.sync_copy(b0.at[pl.ds(0, sz)],
                            acc.at[pl.ds(sid * rpt + off, sz)])
        plsc.subcore_barrier()

        hp2 = rps // 2
        for ph in range(2):
            pltpu.sync_copy(src_hbm.at[pl.ds(wid * rps + ph * hp2, hp2)], idxs)
            pltpu.sync_copy(dst_hbm.at[pl.ds(wid * rps + ph * hp2, hp2)], idxd)

            @pl.loop(0, hp2 // nbuf)
            def _edges(k):
                j = k * nbuf
                gh = [pltpu.async_copy(vals_hbm.at[idxs.at[j + b]],
                                       bufs[b], gsem)
                      for b in range(nbuf)]
                for b in range(nbuf):
                    gh[b].wait()
                    pltpu.sync_copy(bufs[b], acc.at[idxd.at[j + b]], add=True)

        plsc.subcore_barrier()
        for off, sz in _chunks(rpt):
            pltpu.sync_copy(acc.at[pl.ds(sid * rpt + off, sz)],
                            b0.at[pl.ds(0, sz)])
            pltpu.sync_copy(
                b0.at[pl.ds(0, sz)],
                out_hbm.at[pl.ds(cid * npad + sid * rpt + off, sz)])

    return run(vals, src2, dst2).reshape(_NC, npad)


def _seghist(dst2, rps, n):
    rpt = _rpt(n)
    npad = rpt * _NS

    @functools.partial(
        pl.kernel,
        out_type=jax.ShapeDtypeStruct((_NC * npad,), jnp.float32),
        mesh=_sc_mesh(),
        scratch_types=[
            pltpu.VMEM((rps, 128), jnp.int32),
            pltpu.VMEM((128,), jnp.float32),
            pltpu.VMEM_SHARED((npad,), jnp.float32),
        ],
    )
    def run(dst_hbm, out_hbm, idxd, vc, acc):
        cid = lax.axis_index("c")
        sid = lax.axis_index("s")
        wid = cid * _NS + sid

        for j in range(8):
            vc[pl.ds(j * _L, _L)] = jnp.zeros((_L,), jnp.float32)
        for off, sz in _chunks(rpt):
            pltpu.sync_copy(vc.at[pl.ds(0, sz)],
                            acc.at[pl.ds(sid * rpt + off, sz)])
        plsc.subcore_barrier()

        for j in range(8):
            vc[pl.ds(j * _L, _L)] = jnp.ones((_L,), jnp.float32)
        pltpu.sync_copy(dst_hbm.at[pl.ds(wid * rps, rps)], idxd)

        @pl.loop(0, rps)
        def _edges(j):
            pltpu.sync_copy(vc, acc.at[idxd.at[j]], add=True)

        plsc.subcore_barrier()
        for j in range(8):
            vc[pl.ds(j * _L, _L)] = jnp.zeros((_L,), jnp.float32)
        for off, sz in _chunks(rpt):
            pltpu.sync_copy(acc.at[pl.ds(sid * rpt + off, sz)],
                            vc.at[pl.ds(0, sz)])
            pltpu.sync_copy(
                vc.at[pl.ds(0, sz)],
                out_hbm.at[pl.ds(cid * npad + sid * rpt + off, sz)])

    return run(dst2).reshape(_NC, npad)


def _place(v16):
    p = (lax.broadcasted_iota(jnp.int32, (_F, 128), 0)
         == lax.broadcasted_iota(jnp.int32, (_F, 128), 1)).astype(jnp.float32)
    return jnp.dot(v16, p, preferred_element_type=jnp.float32)


def _tc_a(x, w1, degp):
    n, d = x.shape
    f = w1.shape[1]
    rb = 1000
    g = n // rb

    def body(x_ref, w_ref, dp_ref, h_ref, hp_ref, dinv_ref):
        deg = dp_ref[0] + dp_ref[1] + 1.0
        dinv = lax.rsqrt(deg)
        h = jnp.dot(x_ref[...], w_ref[...], preferred_element_type=jnp.float32)
        h_ref[...] = h
        hp_ref[...] = _place(h * dinv)
        dinv_ref[...] = dinv

    return pl.pallas_call(
        body,
        grid=(g,),
        in_specs=[
            pl.BlockSpec((rb, d), lambda i: (i, 0)),
            pl.BlockSpec((d, f), lambda i: (0, 0)),
            pl.BlockSpec((2, rb, 1), lambda i: (0, i, 0)),
        ],
        out_specs=[
            pl.BlockSpec((rb, f), lambda i: (i, 0)),
            pl.BlockSpec((rb, 128), lambda i: (i, 0)),
            pl.BlockSpec((rb, 1), lambda i: (i, 0)),
        ],
        out_shape=[
            jax.ShapeDtypeStruct((n, f), jnp.float32),
            jax.ShapeDtypeStruct((n, 128), jnp.float32),
            jax.ShapeDtypeStruct((n, 1), jnp.float32),
        ],
    )(x, w1, degp)


def _tc_c(s1p, dinv1, h, b1r, wnr, wrr):
    n, f = h.shape
    rb = 1000
    g = n // rb

    def body(s_ref, dinv_ref, h_ref, b1_ref, wn_ref, wr_ref,
             h1_ref, u_ref, v_ref):
        dinv = dinv_ref[...]
        agg = dinv * (s_ref[0] + s_ref[1]) + dinv * dinv * h_ref[...] + b1_ref[...]
        h1 = jnp.maximum(agg, 0.0)
        h1_ref[...] = h1
        u_ref[...] = jnp.sum(h1 * wn_ref[...], axis=1, keepdims=True)
        v_ref[...] = jnp.sum(h1 * wr_ref[...], axis=1, keepdims=True)

    return pl.pallas_call(
        body,
        grid=(g,),
        in_specs=[
            pl.BlockSpec((2, rb, f), lambda i: (0, i, 0)),
            pl.BlockSpec((rb, 1), lambda i: (i, 0)),
            pl.BlockSpec((rb, f), lambda i: (i, 0)),
            pl.BlockSpec((1, f), lambda i: (0, 0)),
            pl.BlockSpec((1, f), lambda i: (0, 0)),
            pl.BlockSpec((1, f), lambda i: (0, 0)),
        ],
        out_specs=[
            pl.BlockSpec((rb, f), lambda i: (i, 0)),
            pl.BlockSpec((rb, 1), lambda i: (i, 0)),
            pl.BlockSpec((rb, 1), lambda i: (i, 0)),
        ],
        out_shape=[
            jax.ShapeDtypeStruct((n, f), jnp.float32),
            jax.ShapeDtypeStruct((n, 1), jnp.float32),
            jax.ShapeDtypeStruct((n, 1), jnp.float32),
        ],
    )(s1p, dinv1, h, b1r, wnr, wrr)


def _tc_d(sac, sar, vcol, vrow, bcol, brow, h1, bsr, nb):
    n, f = h1.shape
    rb = 400
    g = n // rb
    cj = 1000
    ncj = n // cj

    def body(sac_ref, sar_ref, vc_ref, vr_ref, bc_ref, br_ref, h1_ref,
             bs_ref, y_ref, kf_ref, kv_ref):
        i = pl.program_id(0)
        bsv = bs_ref[...]
        score_i = sac_ref[0] + sac_ref[1] + vc_ref[...] + bsv
        srow = sar_ref[0] + sar_ref[1] + vr_ref[...] + bsv
        brow_v = br_ref[...]
        bcol_v = bc_ref[...]

        iota_g = lax.broadcasted_iota(jnp.int32, (nb, 1), 0)
        counts = jnp.sum((brow_v == iota_g).astype(jnp.float32),
                         axis=1, keepdims=True)
        kv = jnp.ceil(jnp.float32(0.8) * counts)
        kv_ref[...] = kv

        iidx = i * rb + lax.broadcasted_iota(jnp.int32, (rb, 1), 0)
        rank = jnp.zeros((rb, 1), jnp.float32)
        for c in range(ncj):
            sj = srow[:, c * cj:(c + 1) * cj]
            bj = brow_v[:, c * cj:(c + 1) * cj]
            jidx = c * cj + lax.broadcasted_iota(jnp.int32, (1, cj), 1)
            hit = ((bj == bcol_v)
                   & ((sj > score_i)
                      | ((sj == score_i) & (jidx < iidx))))
            rank = rank + jnp.sum(hit.astype(jnp.float32),
                                  axis=1, keepdims=True)

        ohr = (bcol_v == lax.broadcasted_iota(jnp.int32, (1, nb), 1)
               ).astype(jnp.float32)
        kb = jnp.dot(ohr, kv, preferred_element_type=jnp.float32)
        kf = (rank < kb).astype(jnp.float32)
        kf_ref[...] = kf
        y_ref[...] = kf * h1_ref[...] * jnp.tanh(score_i)

    return pl.pallas_call(
        body,
        grid=(g,),
        in_specs=[
            pl.BlockSpec((2, rb, 1), lambda i: (0, i, 0)),
            pl.BlockSpec((2, 1, n), lambda i: (0, 0, 0)),
            pl.BlockSpec((rb, 1), lambda i: (i, 0)),
            pl.BlockSpec((1, n), lambda i: (0, 0)),
            pl.BlockSpec((rb, 1), lambda i: (i, 0)),
            pl.BlockSpec((1, n), lambda i: (0, 0)),
            pl.BlockSpec((rb, f), lambda i: (i, 0)),
            pl.BlockSpec((1, 1), lambda i: (0, 0)),
        ],
        out_specs=[
            pl.BlockSpec((rb, f), lambda i: (i, 0)),
            pl.BlockSpec((rb, 1), lambda i: (i, 0)),
            pl.BlockSpec((nb, 1), lambda i: (0, 0)),
        ],
        out_shape=[
            jax.ShapeDtypeStruct((n, f), jnp.float32),
            jax.ShapeDtypeStruct((n, 1), jnp.float32),
            jax.ShapeDtypeStruct((nb, 1), jnp.float32),
        ],
    )(sac, sar, vcol, vrow, bcol, brow, h1, bsr)


def _tc_e(d2p, y):
    n, f = y.shape
    rb = 1000
    g = n // rb

    def body(dp_ref, y_ref, dinv_ref, yp_ref):
        deg = dp_ref[0] + dp_ref[1] + 1.0
        dinv = lax.rsqrt(deg)
        dinv_ref[...] = dinv
        yp_ref[...] = _place(y_ref[...] * dinv)

    return pl.pallas_call(
        body,
        grid=(g,),
        in_specs=[
            pl.BlockSpec((2, rb, 1), lambda i: (0, i, 0)),
            pl.BlockSpec((rb, f), lambda i: (i, 0)),
        ],
        out_specs=[
            pl.BlockSpec((rb, 1), lambda i: (i, 0)),
            pl.BlockSpec((rb, 128), lambda i: (i, 0)),
        ],
        out_shape=[
            jax.ShapeDtypeStruct((n, 1), jnp.float32),
            jax.ShapeDtypeStruct((n, 128), jnp.float32),
        ],
    )(d2p, y)


def _tc_f(s2p, dinv2, y, keptf, w2, b2r, brow, kv, nb):
    n, f = y.shape
    hid = w2.shape[1]
    rb = 1000
    g = n // rb

    def body(s2_ref, dinv_ref, y_ref, kf_ref, w2_ref, b2_ref, br_ref,
             kv_ref, out_ref):
        i = pl.program_id(0)
        dinv = dinv_ref[...]
        agg16 = dinv * (s2_ref[0] + s2_ref[1]) + dinv * dinv * y_ref[...]
        pre = jnp.dot(agg16, w2_ref[...],
                      preferred_element_type=jnp.float32) + b2_ref[...]
        h2 = kf_ref[...] * jnp.maximum(pre, 0.0)
        oh = (br_ref[...] == lax.broadcasted_iota(jnp.int32, (1, nb), 1)
              ).astype(jnp.float32)
        contrib = lax.dot_general(
            oh, h2, (((0,), (0,)), ((), ())),
            preferred_element_type=jnp.float32)

        @pl.when(i == 0)
        def _():
            out_ref[...] = jnp.zeros_like(out_ref)

        out_ref[...] += contrib

        @pl.when(i == g - 1)
        def _():
            out_ref[...] = out_ref[...] / jnp.maximum(kv_ref[...], 1.0)

    return pl.pallas_call(
        body,
        grid=(g,),
        in_specs=[
            pl.BlockSpec((2, rb, f), lambda i: (0, i, 0)),
            pl.BlockSpec((rb, 1), lambda i: (i, 0)),
            pl.BlockSpec((rb, f), lambda i: (i, 0)),
            pl.BlockSpec((rb, 1), lambda i: (i, 0)),
            pl.BlockSpec((f, hid), lambda i: (0, 0)),
            pl.BlockSpec((1, hid), lambda i: (0, 0)),
            pl.BlockSpec((rb, 1), lambda i: (i, 0)),
            pl.BlockSpec((nb, 1), lambda i: (0, 0)),
        ],
        out_specs=pl.BlockSpec((nb, hid), lambda i: (0, 0)),
        out_shape=jax.ShapeDtypeStruct((nb, hid), jnp.float32),
    )(s2p, dinv2, y, keptf, w2, b2r, brow, kv)


def kernel(x, edge_index, batch, W1, b1, Wr, Wn, bs, W2, b2):
    n = x.shape[0]
    nb = 64
    f = W1.shape[1]
    hid = W2.shape[1]
    src2, dst2, rps = _pad_edges(edge_index[0], edge_index[1], n)

    deg1p = _seghist(dst2, rps, n)[:, :n]
    h, hp128, dinv1 = _tc_a(x, W1, deg1p.reshape(2, n, 1))

    s1p = _seg128(hp128, src2, dst2, rps)[:, :n, :f]
    h1, u, v = _tc_c(s1p, dinv1, h, b1.reshape(1, f),
                     Wn.reshape(1, f), Wr.reshape(1, f))

    sap = _segsc(u.reshape(n), src2, dst2, rps)[:, :n]
    y, keptf, kv = _tc_d(sap.reshape(2, n, 1), sap.reshape(2, 1, n),
                         v, v.reshape(1, n),
                         batch.reshape(n, 1), batch.reshape(1, n),
                         h1, bs.reshape(1, 1), nb)

    d2p = _segsc(keptf.reshape(n), src2, dst2, rps)[:, :n]
    dinv2, yp128 = _tc_e(d2p.reshape(2, n, 1), y)

    s2p = _seg128(yp128, src2, dst2, rps)[:, :n, :f]
    return _tc_f(s2p, dinv2, y, keptf, W2, b2.reshape(1, hid),
                 batch.reshape(n, 1), kv, nb)
